# baseline (speedup 1.0000x reference)
import functools

import jax
import jax.numpy as jnp
from jax import lax
from jax.experimental import pallas as pl
from jax.experimental.pallas import tpu as pltpu
from jax.experimental.pallas import tpu_sc as plsc

N = 10000
E = 320000
D = 128
H = 128
G = 64

NC = 2
NS = 16
NW = NC * NS
EPW = E // NW
EK = 80
NCH = EPW // EK
ZW = 10
ZR = 40
ZCH = N // ZW // ZR


def _agg_body(h_hbm, zero_hbm, pk_hbm, out_hbm, pk_all, s0, d0, s1, d1, s2, d2,
              buf0, buf1, buf2, acc_sh, semi, sem0, sem1, sem2,
              ssem0, ssem1, ssem2):
    c = lax.axis_index("c")
    s = lax.axis_index("s")
    wid = s * NC + c

    pltpu.async_copy(pk_hbm.at[wid], pk_all, semi)

    @pl.when(s < ZW)
    def _init_acc():
        r0 = s * (N // ZW)
        nr = N // ZW

        @pl.when(c == 0)
        def _():
            pltpu.sync_copy(h_hbm.at[pl.ds(r0, nr)], acc_sh.at[pl.ds(r0, nr)])

        @pl.when(c == 1)
        def _():
            pltpu.sync_copy(zero_hbm.at[pl.ds(r0, nr)], acc_sh.at[pl.ds(r0, nr)])

    pltpu.make_async_copy(pk_hbm.at[wid], pk_all, semi).wait()
    plsc.subcore_barrier()

    def unpack(i, sbuf, dbuf):
        for v in range(EK // 16):
            p = pk_all[i, pl.ds(v * 16, 16)]
            sbuf[pl.ds(v * 16, 16)] = lax.shift_right_logical(p, 16)
            dbuf[pl.ds(v * 16, 16)] = lax.bitwise_and(p, 0xFFFF)

    sbufs = (s0, s1, s2)
    dbufs = (d0, d1, d2)
    bufs = (buf0, buf1, buf2)
    sems = (sem0, sem1, sem2)
    ssems = (ssem0, ssem1, ssem2)
    unpack(0, s0, d0)
    pltpu.async_copy(h_hbm.at[s0], buf0, sem0)
    unpack(1, s1, d1)
    pltpu.async_copy(h_hbm.at[s1], buf1, sem1)

    def body(j, carry):
        for k in range(3):
            i = 3 * j + k
            b2 = (k + 2) % 3

            @pl.when(i + 2 < NCH)
            def _next():
                @pl.when(i >= 1)
                def _drain():
                    pltpu.make_async_copy(
                        bufs[b2], acc_sh.at[dbufs[b2]], ssems[b2]).wait()

                unpack(i + 2, sbufs[b2], dbufs[b2])
                pltpu.async_copy(h_hbm.at[sbufs[b2]], bufs[b2], sems[b2])

            pltpu.make_async_copy(h_hbm.at[sbufs[k]], bufs[k], sems[k]).wait()
            pltpu.async_copy(bufs[k], acc_sh.at[dbufs[k]], ssems[k], add=True)
        return carry

    lax.fori_loop(0, NCH // 3, body, 0)
    for i in range(NCH - NCH % 3, NCH):
        k = i % 3
        pltpu.make_async_copy(h_hbm.at[sbufs[k]], bufs[k], sems[k]).wait()
        pltpu.async_copy(bufs[k], acc_sh.at[dbufs[k]], ssems[k], add=True)
    for i in range(NCH - 3, NCH):
        k = i % 3
        pltpu.make_async_copy(bufs[k], acc_sh.at[dbufs[k]], ssems[k]).wait()
    plsc.subcore_barrier()

    @pl.when(s < ZW)
    def _writeback():
        r0 = s * (N // ZW)
        nr = N // ZW
        pltpu.sync_copy(acc_sh.at[pl.ds(r0, nr)], out_hbm.at[c, pl.ds(r0, nr)])


@functools.lru_cache(maxsize=None)
def _build_agg():
    mesh = plsc.VectorSubcoreMesh(core_axis_name="c", subcore_axis_name="s")
    return pl.kernel(
        _agg_body,
        out_type=jax.ShapeDtypeStruct((NC, N, D), jnp.float32),
        mesh=mesh,
        scratch_types=[
            pltpu.VMEM((NCH, EK), jnp.int32),
            pltpu.VMEM((EK,), jnp.int32),
            pltpu.VMEM((EK,), jnp.int32),
            pltpu.VMEM((EK,), jnp.int32),
            pltpu.VMEM((EK,), jnp.int32),
            pltpu.VMEM((EK,), jnp.int32),
            pltpu.VMEM((EK,), jnp.int32),
            pltpu.VMEM((EK, D), jnp.float32),
            pltpu.VMEM((EK, D), jnp.float32),
            pltpu.VMEM((EK, D), jnp.float32),
            pltpu.VMEM_SHARED((N, D), jnp.float32),
            pltpu.SemaphoreType.DMA,
            pltpu.SemaphoreType.DMA,
            pltpu.SemaphoreType.DMA,
            pltpu.SemaphoreType.DMA,
            pltpu.SemaphoreType.DMA,
            pltpu.SemaphoreType.DMA,
            pltpu.SemaphoreType.DMA,
        ],
    )


_BR = 2000


def _mlp_body(a0_ref, a1_ref, w1_ref, b1_ref, w2_ref, b2_ref, o_ref):
    z = a0_ref[0] + a1_ref[0]
    y = jnp.dot(z, w1_ref[...], preferred_element_type=jnp.float32) + b1_ref[...]
    y = jnp.maximum(y, 0.0)
    o = jnp.dot(y, w2_ref[...], preferred_element_type=jnp.float32) + b2_ref[...]
    o_ref[...] = jnp.maximum(o, 0.0)


def _mlp(parts, w1, b1, w2, b2):
    grid = N // _BR
    return pl.pallas_call(
        _mlp_body,
        grid=(grid,),
        in_specs=[
            pl.BlockSpec((1, _BR, D), lambda i: (0, i, 0)),
            pl.BlockSpec((1, _BR, D), lambda i: (1, i, 0)),
            pl.BlockSpec((D, H), lambda i: (0, 0)),
            pl.BlockSpec((1, H), lambda i: (0, 0)),
            pl.BlockSpec((H, H), lambda i: (0, 0)),
            pl.BlockSpec((1, H), lambda i: (0, 0)),
        ],
        out_specs=pl.BlockSpec((_BR, H), lambda i: (i, 0)),
        out_shape=jax.ShapeDtypeStruct((N, H), jnp.float32),
    )(parts, parts, w1, b1, w2, b2)


def _mlp_pool_body(a0_ref, a1_ref, w1_ref, b1_ref, w2_ref, b2_ref,
                   batch_ref, o_ref, sums_ref, cnt_ref):
    i = pl.program_id(0)
    z = a0_ref[0] + a1_ref[0]
    y = jnp.dot(z, w1_ref[...], preferred_element_type=jnp.float32) + b1_ref[...]
    y = jnp.maximum(y, 0.0)
    o = jnp.dot(y, w2_ref[...], preferred_element_type=jnp.float32) + b2_ref[...]
    h3 = jnp.maximum(o, 0.0)

    seg = lax.broadcasted_iota(jnp.int32, (_BR, G), 1)
    m = (batch_ref[...] == seg).astype(jnp.float32)
    dn = (((0,), (0,)), ((), ()))
    psum = lax.dot_general(m, h3, dn, preferred_element_type=jnp.float32)
    pcnt = lax.dot_general(m, jnp.ones_like(h3), dn,
                           preferred_element_type=jnp.float32)

    @pl.when(i == 0)
    def _init():
        sums_ref[...] = jnp.zeros_like(sums_ref)
        cnt_ref[...] = jnp.zeros_like(cnt_ref)

    sums_ref[...] += psum
    cnt_ref[...] += pcnt

    @pl.when(i == pl.num_programs(0) - 1)
    def _fin():
        o_ref[...] = sums_ref[...] / jnp.maximum(cnt_ref[...], 1.0)


def _mlp_pool(parts, w1, b1, w2, b2, batch2d):
    grid = N // _BR
    return pl.pallas_call(
        _mlp_pool_body,
        grid=(grid,),
        in_specs=[
            pl.BlockSpec((1, _BR, D), lambda i: (0, i, 0)),
            pl.BlockSpec((1, _BR, D), lambda i: (1, i, 0)),
        ] + [
            pl.BlockSpec((D, H), lambda i: (0, 0)),
            pl.BlockSpec((1, H), lambda i: (0, 0)),
            pl.BlockSpec((H, H), lambda i: (0, 0)),
            pl.BlockSpec((1, H), lambda i: (0, 0)),
            pl.BlockSpec((_BR, 1), lambda i: (i, 0)),
        ],
        out_specs=pl.BlockSpec((G, H), lambda i: (0, 0)),
        out_shape=jax.ShapeDtypeStruct((G, H), jnp.float32),
        scratch_shapes=[
            pltpu.VMEM((G, H), jnp.float32),
            pltpu.VMEM((G, H), jnp.float32),
        ],
    )(parts, parts, w1, b1, w2, b2, batch2d)


def kernel(x, edge_index, batch, w1_0, b1_0, w2_0, b2_0, w1_1, b1_1, w2_1,
           b2_1, w1_2, b1_2, w2_2, b2_2):
    packed = ((edge_index[0] << 16) | edge_index[1]).reshape(NW, NCH, EK)
    batch2d = batch.reshape(N, 1)
    params = [(w1_0, b1_0, w2_0, b2_0), (w1_1, b1_1, w2_1, b2_1),
              (w1_2, b1_2, w2_2, b2_2)]
    agg = _build_agg()
    zeros = jnp.zeros((N, D), jnp.float32)
    h = x
    out = None
    for li, (w1, b1, w2, b2) in enumerate(params):
        parts = agg(h, zeros, packed)
        b1r = b1.reshape(1, H)
        b2r = b2.reshape(1, H)
        if li < 2:
            h = _mlp(parts, w1, b1r, w2, b2r)
        else:
            out = _mlp_pool(parts, w1, b1r, w2, b2r, batch2d)
    return out


# speedup vs baseline: 1.0049x; 1.0049x over previous
import functools

import jax
import jax.numpy as jnp
from jax import lax
from jax.experimental import pallas as pl
from jax.experimental.pallas import tpu as pltpu
from jax.experimental.pallas import tpu_sc as plsc

N = 10000
E = 320000
D = 128
H = 128
G = 64

NC = 2
NS = 16
NW = NC * NS
EPW = E // NW
EK = 80
NCH = EPW // EK
ZW = 10
ZR = 40
ZCH = N // ZW // ZR


def _agg_body(h_hbm, zero_hbm, pk_hbm, out_hbm, pk_all, s0, d0, s1, d1, s2, d2,
              buf0, buf1, buf2, acc_sh, semi, sem0, sem1, sem2,
              ssem0, ssem1, ssem2):
    c = lax.axis_index("c")
    s = lax.axis_index("s")
    wid = s * NC + c

    pltpu.async_copy(pk_hbm.at[wid], pk_all, semi)

    @pl.when(s < ZW)
    def _init_acc():
        r0 = s * (N // ZW)
        nr = N // ZW

        @pl.when(c == 0)
        def _():
            pltpu.sync_copy(h_hbm.at[pl.ds(r0, nr)], acc_sh.at[pl.ds(r0, nr)])

        @pl.when(c == 1)
        def _():
            pltpu.sync_copy(zero_hbm.at[pl.ds(r0, nr)], acc_sh.at[pl.ds(r0, nr)])

    pltpu.make_async_copy(pk_hbm.at[wid], pk_all, semi).wait()

    def unpack(i, sbuf, dbuf):
        for v in range(EK // 16):
            p = pk_all[i, pl.ds(v * 16, 16)]
            sbuf[pl.ds(v * 16, 16)] = lax.shift_right_logical(p, 16)
            dbuf[pl.ds(v * 16, 16)] = lax.bitwise_and(p, 0xFFFF)

    sbufs = (s0, s1, s2)
    dbufs = (d0, d1, d2)
    bufs = (buf0, buf1, buf2)
    sems = (sem0, sem1, sem2)
    ssems = (ssem0, ssem1, ssem2)
    unpack(0, s0, d0)
    pltpu.async_copy(h_hbm.at[s0], buf0, sem0)
    unpack(1, s1, d1)
    pltpu.async_copy(h_hbm.at[s1], buf1, sem1)
    plsc.subcore_barrier()

    def body(j, carry):
        for k in range(3):
            i = 3 * j + k
            b2 = (k + 2) % 3

            @pl.when(i + 2 < NCH)
            def _next():
                @pl.when(i >= 1)
                def _drain():
                    pltpu.make_async_copy(
                        bufs[b2], acc_sh.at[dbufs[b2]], ssems[b2]).wait()

                unpack(i + 2, sbufs[b2], dbufs[b2])
                pltpu.async_copy(h_hbm.at[sbufs[b2]], bufs[b2], sems[b2])

            pltpu.make_async_copy(h_hbm.at[sbufs[k]], bufs[k], sems[k]).wait()
            pltpu.async_copy(bufs[k], acc_sh.at[dbufs[k]], ssems[k], add=True)
        return carry

    lax.fori_loop(0, NCH // 3, body, 0)
    for i in range(NCH - NCH % 3, NCH):
        k = i % 3
        pltpu.make_async_copy(h_hbm.at[sbufs[k]], bufs[k], sems[k]).wait()
        pltpu.async_copy(bufs[k], acc_sh.at[dbufs[k]], ssems[k], add=True)
    for i in range(NCH - 3, NCH):
        k = i % 3
        pltpu.make_async_copy(bufs[k], acc_sh.at[dbufs[k]], ssems[k]).wait()
    plsc.subcore_barrier()

    @pl.when(s < ZW)
    def _writeback():
        r0 = s * (N // ZW)
        nr = N // ZW
        pltpu.sync_copy(acc_sh.at[pl.ds(r0, nr)], out_hbm.at[c, pl.ds(r0, nr)])


@functools.lru_cache(maxsize=None)
def _build_agg():
    mesh = plsc.VectorSubcoreMesh(core_axis_name="c", subcore_axis_name="s")
    return pl.kernel(
        _agg_body,
        out_type=jax.ShapeDtypeStruct((NC, N, D), jnp.float32),
        mesh=mesh,
        scratch_types=[
            pltpu.VMEM((NCH, EK), jnp.int32),
            pltpu.VMEM((EK,), jnp.int32),
            pltpu.VMEM((EK,), jnp.int32),
            pltpu.VMEM((EK,), jnp.int32),
            pltpu.VMEM((EK,), jnp.int32),
            pltpu.VMEM((EK,), jnp.int32),
            pltpu.VMEM((EK,), jnp.int32),
            pltpu.VMEM((EK, D), jnp.float32),
            pltpu.VMEM((EK, D), jnp.float32),
            pltpu.VMEM((EK, D), jnp.float32),
            pltpu.VMEM_SHARED((N, D), jnp.float32),
            pltpu.SemaphoreType.DMA,
            pltpu.SemaphoreType.DMA,
            pltpu.SemaphoreType.DMA,
            pltpu.SemaphoreType.DMA,
            pltpu.SemaphoreType.DMA,
            pltpu.SemaphoreType.DMA,
            pltpu.SemaphoreType.DMA,
        ],
    )


_BR = 2000


def _mlp_body(a0_ref, a1_ref, w1_ref, b1_ref, w2_ref, b2_ref, o_ref):
    z = a0_ref[0] + a1_ref[0]
    y = jnp.dot(z, w1_ref[...], preferred_element_type=jnp.float32) + b1_ref[...]
    y = jnp.maximum(y, 0.0)
    o = jnp.dot(y, w2_ref[...], preferred_element_type=jnp.float32) + b2_ref[...]
    o_ref[...] = jnp.maximum(o, 0.0)


def _mlp(parts, w1, b1, w2, b2):
    grid = N // _BR
    return pl.pallas_call(
        _mlp_body,
        grid=(grid,),
        in_specs=[
            pl.BlockSpec((1, _BR, D), lambda i: (0, i, 0)),
            pl.BlockSpec((1, _BR, D), lambda i: (1, i, 0)),
            pl.BlockSpec((D, H), lambda i: (0, 0)),
            pl.BlockSpec((1, H), lambda i: (0, 0)),
            pl.BlockSpec((H, H), lambda i: (0, 0)),
            pl.BlockSpec((1, H), lambda i: (0, 0)),
        ],
        out_specs=pl.BlockSpec((_BR, H), lambda i: (i, 0)),
        out_shape=jax.ShapeDtypeStruct((N, H), jnp.float32),
    )(parts, parts, w1, b1, w2, b2)


def _mlp_pool_body(a0_ref, a1_ref, w1_ref, b1_ref, w2_ref, b2_ref,
                   batch_ref, o_ref, sums_ref, cnt_ref):
    i = pl.program_id(0)
    z = a0_ref[0] + a1_ref[0]
    y = jnp.dot(z, w1_ref[...], preferred_element_type=jnp.float32) + b1_ref[...]
    y = jnp.maximum(y, 0.0)
    o = jnp.dot(y, w2_ref[...], preferred_element_type=jnp.float32) + b2_ref[...]
    h3 = jnp.maximum(o, 0.0)

    seg = lax.broadcasted_iota(jnp.int32, (_BR, G), 1)
    m = (batch_ref[...] == seg).astype(jnp.float32)
    dn = (((0,), (0,)), ((), ()))
    psum = lax.dot_general(m, h3, dn, preferred_element_type=jnp.float32)
    pcnt = lax.dot_general(m, jnp.ones_like(h3), dn,
                           preferred_element_type=jnp.float32)

    @pl.when(i == 0)
    def _init():
        sums_ref[...] = jnp.zeros_like(sums_ref)
        cnt_ref[...] = jnp.zeros_like(cnt_ref)

    sums_ref[...] += psum
    cnt_ref[...] += pcnt

    @pl.when(i == pl.num_programs(0) - 1)
    def _fin():
        o_ref[...] = sums_ref[...] / jnp.maximum(cnt_ref[...], 1.0)


def _mlp_pool(parts, w1, b1, w2, b2, batch2d):
    grid = N // _BR
    return pl.pallas_call(
        _mlp_pool_body,
        grid=(grid,),
        in_specs=[
            pl.BlockSpec((1, _BR, D), lambda i: (0, i, 0)),
            pl.BlockSpec((1, _BR, D), lambda i: (1, i, 0)),
        ] + [
            pl.BlockSpec((D, H), lambda i: (0, 0)),
            pl.BlockSpec((1, H), lambda i: (0, 0)),
            pl.BlockSpec((H, H), lambda i: (0, 0)),
            pl.BlockSpec((1, H), lambda i: (0, 0)),
            pl.BlockSpec((_BR, 1), lambda i: (i, 0)),
        ],
        out_specs=pl.BlockSpec((G, H), lambda i: (0, 0)),
        out_shape=jax.ShapeDtypeStruct((G, H), jnp.float32),
        scratch_shapes=[
            pltpu.VMEM((G, H), jnp.float32),
            pltpu.VMEM((G, H), jnp.float32),
        ],
    )(parts, parts, w1, b1, w2, b2, batch2d)


def kernel(x, edge_index, batch, w1_0, b1_0, w2_0, b2_0, w1_1, b1_1, w2_1,
           b2_1, w1_2, b1_2, w2_2, b2_2):
    packed = ((edge_index[0] << 16) | edge_index[1]).reshape(NW, NCH, EK)
    batch2d = batch.reshape(N, 1)
    params = [(w1_0, b1_0, w2_0, b2_0), (w1_1, b1_1, w2_1, b2_1),
              (w1_2, b1_2, w2_2, b2_2)]
    agg = _build_agg()
    zeros = jnp.zeros((N, D), jnp.float32)
    h = x
    out = None
    for li, (w1, b1, w2, b2) in enumerate(params):
        parts = agg(h, zeros, packed)
        b1r = b1.reshape(1, H)
        b2r = b2.reshape(1, H)
        if li < 2:
            h = _mlp(parts, w1, b1r, w2, b2r)
        else:
            out = _mlp_pool(parts, w1, b1r, w2, b2r, batch2d)
    return out
